# baseline (speedup 1.0000x reference)
import functools
import math

import jax
import jax.numpy as jnp
from jax import lax
from jax.experimental import pallas as pl
from jax.experimental.pallas import tpu as pltpu
from jax.experimental.pallas import tpu_sc as plsc

LANES = 16

_info = plsc.get_sparse_core_info()
NUM_CORES = _info.num_cores
NUM_SUBCORES = _info.num_subcores
NUM_WORKERS = NUM_CORES * NUM_SUBCORES

BLK = 128
NR = 4
PF = 2
NW = 2


def _make_lookup(B, S, V, D):
    assert B % BLK == 0 and B // BLK == NUM_WORKERS
    assert S % NR == 0
    scale = math.sqrt(D)
    n_vecs_t = BLK // LANES
    mesh = plsc.VectorSubcoreMesh(core_axis_name="c", subcore_axis_name="s")

    @functools.partial(
        pl.kernel,
        mesh=mesh,
        compiler_params=pltpu.CompilerParams(
            use_tc_tiling_on_sc=True, needs_layout_passes=False
        ),
        out_type=jax.ShapeDtypeStruct((S, D, B), jnp.float32),
        scratch_types=[
            [pltpu.VMEM((1, BLK), jnp.int32) for _ in range(NR)],
            [pltpu.VMEM((BLK,), jnp.int32) for _ in range(NR)],
            [pltpu.VMEM((BLK, 2 * D), jnp.float32) for _ in range(NR)],
            [pltpu.VMEM((1, D, BLK), jnp.float32) for _ in range(NW)],
            [pltpu.SemaphoreType.DMA for _ in range(NR)],
            [pltpu.SemaphoreType.DMA for _ in range(NW)],
        ],
    )
    def lookup(xt_hbm, lut2_hbm, out_hbm, idxs, hidxs, rows, slabs, gsem, ssem):
        wid = lax.axis_index("s") * NUM_CORES + lax.axis_index("c")
        bbase = wid * BLK

        def stage_idx(k, r):
            pltpu.sync_copy(xt_hbm.at[pl.ds(k, 1), pl.ds(bbase, BLK)], idxs[r])
            for t in range(n_vecs_t):
                sl = pl.ds(t * LANES, LANES)
                hidxs[r][sl] = lax.shift_right_logical(idxs[r][0, sl], 1)

        def gather_copy(r):
            return pltpu.make_async_copy(lut2_hbm.at[hidxs[r]], rows[r], gsem[r])

        def slab_copy(k, c):
            return pltpu.make_async_copy(
                slabs[c], out_hbm.at[pl.ds(k, 1), :, pl.ds(bbase, BLK)], ssem[c]
            )

        for k0 in range(PF):
            stage_idx(k0, k0)
            gather_copy(k0).start()

        def step(k, r, c, can_pref, need_wait):
            pr = (r + PF) % NR

            @pl.when(can_pref)
            def _():
                stage_idx(k + PF, pr)
                gather_copy(pr).start()

            gather_copy(r).wait()

            @pl.when(need_wait)
            def _():
                slab_copy(k - NW, c).wait()

            rb = rows[r]
            sb = slabs[c]
            i_vecs = []
            j_vecs = []
            for t in range(n_vecs_t):
                sl = pl.ds(t * LANES, LANES)
                i_vecs.append(lax.iota(jnp.int32, LANES) + t * LANES)
                j_vecs.append((idxs[r][0, sl] & 1) * D)

            @plsc.parallel_loop(0, D, step=2, unroll=1)
            def _(d):
                for dd in range(2):
                    for t in range(n_vecs_t):
                        v = plsc.load_gather(rb, [i_vecs[t], j_vecs[t] + (d + dd)])
                        sb[0, d + dd, pl.ds(t * LANES, LANES)] = v * scale

            slab_copy(k, c).start()

        def group_body(g, carry):
            for j in range(NR):
                k = NR * g + j
                can_pref = (g < S // NR - 1) if j >= NR - PF else jnp.bool_(True)
                need_wait = (g > 0) if j < NW else jnp.bool_(True)
                step(k, j, j % NW, can_pref, need_wait)
            return carry

        lax.fori_loop(0, S // NR, group_body, 0)
        slab_copy(S - 2, 0).wait()
        slab_copy(S - 1, 1).wait()

    return lookup


def kernel(x, lut):
    B, S = x.shape
    V, D = lut.shape
    xt = x.T.astype(jnp.int32)
    lut2 = lut.reshape(V // 2, 2 * D)
    outp = _make_lookup(B, S, V, D)(xt, lut2)
    return outp.transpose(2, 0, 1)


# speedup vs baseline: 1.0248x; 1.0248x over previous
import functools
import math

import jax
import jax.numpy as jnp
from jax import lax
from jax.experimental import pallas as pl
from jax.experimental.pallas import tpu as pltpu
from jax.experimental.pallas import tpu_sc as plsc

LANES = 16

_info = plsc.get_sparse_core_info()
NUM_CORES = _info.num_cores
NUM_SUBCORES = _info.num_subcores
NUM_WORKERS = NUM_CORES * NUM_SUBCORES

CH = 128
NR = 4
PF = 2
NW = 2


def _make_lookup(N, V, D):
    assert N % (NUM_WORKERS * CH) == 0
    per_w = N // NUM_WORKERS
    steps = per_w // CH
    assert steps % NR == 0 and steps >= NR
    scale = math.sqrt(D)
    n_vec = D // LANES
    mesh = plsc.VectorSubcoreMesh(core_axis_name="c", subcore_axis_name="s")

    @functools.partial(
        pl.kernel,
        mesh=mesh,
        compiler_params=pltpu.CompilerParams(
            use_tc_tiling_on_sc=True, needs_layout_passes=False
        ),
        out_type=jax.ShapeDtypeStruct((N // 2, 2 * D), jnp.float32),
        scratch_types=[
            [pltpu.VMEM((CH,), jnp.int32) for _ in range(NR)],
            [pltpu.VMEM((CH,), jnp.int32) for _ in range(NR)],
            [pltpu.VMEM((CH,), jnp.int32) for _ in range(NR)],
            [pltpu.VMEM((CH, 2 * D), jnp.float32) for _ in range(NR)],
            [pltpu.VMEM((CH // 2, 2 * D), jnp.float32) for _ in range(NW)],
            [pltpu.SemaphoreType.DMA for _ in range(NR)],
            [pltpu.SemaphoreType.DMA for _ in range(NW)],
        ],
    )
    def lookup(idx_hbm, lut2_hbm, out_hbm, idxs, hidxs, sels, rows, blks,
               gsem, osem):
        wid = lax.axis_index("s") * NUM_CORES + lax.axis_index("c")
        base = wid * per_w
        iota = lax.iota(jnp.int32, LANES)
        zero = iota * 0

        def stage_idx(k, r):
            pltpu.sync_copy(idx_hbm.at[pl.ds(base + k * CH, CH)], idxs[r])
            for t in range(CH // LANES):
                sl = pl.ds(t * LANES, LANES)
                v = idxs[r][sl]
                hidxs[r][sl] = lax.shift_right_logical(v, 1)
                sels[r][sl] = (v & 1) * D

        def gather_copy(r):
            return pltpu.make_async_copy(lut2_hbm.at[hidxs[r]], rows[r], gsem[r])

        def out_copy(k, c):
            return pltpu.make_async_copy(
                blks[c],
                out_hbm.at[pl.ds(pl.multiple_of((base + k * CH) // 2, 8), CH // 2)],
                osem[c],
            )

        for k0 in range(PF):
            stage_idx(k0, k0)
            gather_copy(k0).start()

        def step(k, r, c, can_pref, need_wait):
            pr = (r + PF) % NR

            @pl.when(can_pref)
            def _():
                stage_idx(k + PF, pr)
                gather_copy(pr).start()

            gather_copy(r).wait()

            @pl.when(need_wait)
            def _():
                out_copy(k - NW, c).wait()

            rb = rows[r]
            sb = sels[r]
            bb = blks[c]

            @plsc.parallel_loop(0, CH // 2, step=1, unroll=1)
            def _(p):
                for ii in range(2):
                    q = 2 * p + ii
                    qv = zero + q
                    off = plsc.load_gather(sb, [qv])
                    for t in range(n_vec):
                        v = plsc.load_gather(rb, [qv, off + (t * LANES + iota)])
                        bb[p, pl.ds(ii * D + t * LANES, LANES)] = v * scale

            out_copy(k, c).start()

        def group_body(g, carry):
            for j in range(NR):
                k = NR * g + j
                can_pref = (g < steps // NR - 1) if j >= NR - PF else jnp.bool_(True)
                need_wait = (g > 0) if j < NW else jnp.bool_(True)
                step(k, j, j % NW, can_pref, need_wait)
            return carry

        lax.fori_loop(0, steps // NR, group_body, 0)
        out_copy(steps - 2, 0).wait()
        out_copy(steps - 1, 1).wait()

    return lookup


def kernel(x, lut):
    B, S = x.shape
    V, D = lut.shape
    idx = x.T.reshape(-1).astype(jnp.int32)
    lut2 = lut.reshape(V // 2, 2 * D)
    out2 = _make_lookup(B * S, V, D)(idx, lut2)
    return out2.reshape(S, B, D).transpose(1, 0, 2)


# speedup vs baseline: 1.0302x; 1.0052x over previous
import functools
import math

import jax
import jax.numpy as jnp
from jax import lax
from jax.experimental import pallas as pl
from jax.experimental.pallas import tpu as pltpu
from jax.experimental.pallas import tpu_sc as plsc

LANES = 16

_info = plsc.get_sparse_core_info()
NUM_CORES = _info.num_cores
NUM_SUBCORES = _info.num_subcores
NUM_WORKERS = NUM_CORES * NUM_SUBCORES

CH = 128
NR = 4
PF = 2
NW = 2


def _make_lookup(N, V, D):
    assert N % (NUM_WORKERS * CH) == 0
    per_w = N // NUM_WORKERS
    steps = per_w // CH
    assert steps % NR == 0 and steps >= NR
    scale = math.sqrt(D)
    n_vec = D // LANES
    mesh = plsc.VectorSubcoreMesh(core_axis_name="c", subcore_axis_name="s")

    @functools.partial(
        pl.kernel,
        mesh=mesh,
        compiler_params=pltpu.CompilerParams(
            use_tc_tiling_on_sc=True, needs_layout_passes=False
        ),
        out_type=jax.ShapeDtypeStruct((N // 2, 2 * D), jnp.float32),
        scratch_types=[
            [pltpu.VMEM((CH,), jnp.int32) for _ in range(NR)],
            [pltpu.VMEM((CH,), jnp.int32) for _ in range(NR)],
            [pltpu.VMEM((CH,), jnp.int32) for _ in range(NR)],
            [pltpu.VMEM((CH, 2 * D), jnp.float32) for _ in range(NR)],
            [pltpu.VMEM((CH // 2, 2 * D), jnp.float32) for _ in range(NW)],
            [pltpu.SemaphoreType.DMA for _ in range(NR)],
            [pltpu.SemaphoreType.DMA for _ in range(NW)],
        ],
    )
    def lookup(idx_hbm, lut2_hbm, out_hbm, idxs, hidxs, sels, rows, blks,
               gsem, osem):
        wid = lax.axis_index("s") * NUM_CORES + lax.axis_index("c")
        base = wid * per_w
        iota = lax.iota(jnp.int32, LANES)
        zero = iota * 0

        def stage_idx(k, r):
            pltpu.sync_copy(idx_hbm.at[pl.ds(base + k * CH, CH)], idxs[r])
            for t in range(CH // LANES):
                sl = pl.ds(t * LANES, LANES)
                v = idxs[r][sl]
                hidxs[r][sl] = lax.shift_right_logical(v, 1)
                sels[r][sl] = (v & 1) * D

        def gather_copy(r):
            return pltpu.make_async_copy(lut2_hbm.at[hidxs[r]], rows[r], gsem[r])

        def out_copy(k, c):
            return pltpu.make_async_copy(
                blks[c],
                out_hbm.at[pl.ds(pl.multiple_of((base + k * CH) // 2, 8), CH // 2)],
                osem[c],
            )

        for k0 in range(PF):
            stage_idx(k0, k0)
            gather_copy(k0).start()

        def step(k, r, c, can_pref, need_wait):
            pr = (r + PF) % NR

            @pl.when(can_pref)
            def _():
                stage_idx(k + PF, pr)
                gather_copy(pr).start()

            gather_copy(r).wait()

            @pl.when(need_wait)
            def _():
                out_copy(k - NW, c).wait()

            rb = rows[r]
            sb = sels[r]
            bb = blks[c]

            @plsc.parallel_loop(0, CH // 2, step=1, unroll=4)
            def _(p):
                for ii in range(2):
                    q = 2 * p + ii
                    qv = zero + q
                    off = plsc.load_gather(sb, [qv])
                    for t in range(n_vec):
                        v = plsc.load_gather(rb, [qv, off + (t * LANES + iota)])
                        bb[p, pl.ds(ii * D + t * LANES, LANES)] = v * scale

            out_copy(k, c).start()

        def group_body(g, carry):
            for j in range(NR):
                k = NR * g + j
                can_pref = (g < steps // NR - 1) if j >= NR - PF else jnp.bool_(True)
                need_wait = (g > 0) if j < NW else jnp.bool_(True)
                step(k, j, j % NW, can_pref, need_wait)
            return carry

        lax.fori_loop(0, steps // NR, group_body, 0)
        out_copy(steps - 2, 0).wait()
        out_copy(steps - 1, 1).wait()

    return lookup


def kernel(x, lut):
    B, S = x.shape
    V, D = lut.shape
    idx = x.T.reshape(-1).astype(jnp.int32)
    lut2 = lut.reshape(V // 2, 2 * D)
    out2 = _make_lookup(B * S, V, D)(idx, lut2)
    return out2.reshape(S, B, D).transpose(1, 0, 2)
